# baseline (speedup 1.0000x reference)
import functools

import jax
import jax.numpy as jnp
from jax import lax
from jax.experimental import pallas as pl
from jax.experimental.pallas import tpu as pltpu
from jax.experimental.pallas import tpu_sc as plsc

_N = 5000
_TOP_K = 100
_SCALE = 192.0
_IOU_T = 0.3
_P = 5120
_PAD = _P - _N
_NS = 16
_SHARD = _P // _NS
_CHUNKS = _SHARD // 16
_WPAD = _SHARD + 16
_GRID = _NS * 16
_BIG = 2 ** 30


def _sc_nms(feat_hbm, out_hbm,
            f_v, s_v, cx_v, cy_v, w_v, p0x_v, p0y_v, p2x_v, p2y_v,
            x1_v, x2_v, yl_v, yh_v, ar_v,
            stage_v, grid_v, out_v, grid_sh):
    cid = lax.axis_index("c")
    wid = lax.axis_index("s")

    @pl.when(cid == 0)
    def _body():
        base = wid * _SHARD

        for j in range(11):
            pltpu.sync_copy(feat_hbm.at[pl.ds(j * _P + base, _SHARD)],
                            f_v.at[pl.ds(j * _SHARD, _SHARD)])

        mv0 = jnp.full((16,), -2.0, jnp.float32)
        iv0 = jnp.zeros((16,), jnp.int32)
        for c in range(_CHUNKS):
            sl = pl.ds(c * 16, 16)
            lane = lax.iota(jnp.int32, 16) + (base + c * 16)
            lg = f_v[pl.ds(0 * _SHARD + c * 16, 16)]
            sc = 1.0 / (1.0 + jnp.exp(-lg))
            ax = f_v[pl.ds(9 * _SHARD + c * 16, 16)]
            ay = f_v[pl.ds(10 * _SHARD + c * 16, 16)]
            cx = f_v[pl.ds(1 * _SHARD + c * 16, 16)] / _SCALE + ax
            cy = f_v[pl.ds(2 * _SHARD + c * 16, 16)] / _SCALE + ay
            w = f_v[pl.ds(3 * _SHARD + c * 16, 16)] / _SCALE
            h = f_v[pl.ds(4 * _SHARD + c * 16, 16)] / _SCALE
            half_w = w * 0.5
            half_h = h * 0.5
            x1 = cx - half_w
            x2 = cx + half_w
            yl = cy - half_h
            yh = cy + half_h
            s0 = jnp.where(lane < _N, sc, -1.0)
            iv0 = jnp.where(s0 > mv0, lane, iv0)
            mv0 = jnp.maximum(mv0, s0)
            s_v[sl] = s0
            cx_v[sl] = cx
            cy_v[sl] = cy
            w_v[sl] = w
            p0x_v[sl] = f_v[pl.ds(5 * _SHARD + c * 16, 16)] / _SCALE + ax
            p0y_v[sl] = f_v[pl.ds(6 * _SHARD + c * 16, 16)] / _SCALE + ay
            p2x_v[sl] = f_v[pl.ds(7 * _SHARD + c * 16, 16)] / _SCALE + ax
            p2y_v[sl] = f_v[pl.ds(8 * _SHARD + c * 16, 16)] / _SCALE + ay
            x1_v[sl] = x1
            x2_v[sl] = x2
            yl_v[sl] = yl
            yh_v[sl] = yh
            ar_v[sl] = (x2 - x1) * (yh - yl)

        l16 = lax.iota(jnp.int32, 16)
        zf = jnp.zeros((16,), jnp.float32)
        zi = jnp.zeros((16,), jnp.int32)

        def _vsel(cond_scalar_f32, a, b):
            return a * cond_scalar_f32 + b * (1.0 - cond_scalar_f32)

        def body(t, carry):
            k0r, mv, iv = carry
            mvv, ivv = mv, iv
            for sh in (1, 2, 4, 8):
                perm = l16 ^ sh
                pm = mvv[perm]
                pi = ivv[perm]
                b = (pm > mvv) | ((pm == mvv) & (pi < ivv))
                mvv = jnp.where(b, pm, mvv)
                ivv = jnp.where(b, pi, ivv)
            gi = ivv[0]
            off = jnp.clip(gi - base, 0, _SHARD - 1)

            gidx_f = (base + off).astype(jnp.float32)
            srcs = (cx_v, cy_v, w_v, p0x_v, p0y_v, p2x_v, p2y_v,
                    x1_v, x2_v, yl_v, yh_v, ar_v)
            stage = jnp.where(l16 == 1, gidx_f, mvv)
            stage = jnp.where(l16 >= 3, 0.0, stage)
            for j, ref in enumerate(srcs):
                stage = jnp.where(l16 == j + 3, ref[pl.ds(off, 16)][0], stage)
            stage_v[...] = stage
            par = (t % 2) * _GRID
            pltpu.sync_copy(stage_v, grid_sh.at[pl.ds(par + wid * 16, 16)])
            plsc.subcore_barrier()
            pltpu.sync_copy(grid_sh.at[pl.ds(par, _GRID)], grid_v)

            items = [(grid_v[pl.ds(j * 16, 16)][0], jnp.int32(j * 16))
                     for j in range(_NS)]
            while len(items) > 1:
                nxt = []
                for (ka, ra), (kb, rb) in zip(items[::2], items[1::2]):
                    b = kb > ka
                    nxt.append((jnp.maximum(ka, kb),
                                jnp.where(b, rb, ra)))
                items = nxt
            _wkey, rwin = items[0]
            w_rec = grid_v[pl.ds(rwin, 16)]

            has_f = (w_rec[0] >= 0.0).astype(jnp.float32)
            t0_f = (t == 0).astype(jnp.float32)
            k0r = _vsel(t0_f, w_rec, k0r)
            wu = _vsel(has_f, w_rec, k0r)

            k = wu[1].astype(jnp.int32)
            kx1 = wu[jnp.full((16,), 10, jnp.int32)]
            kx2 = wu[jnp.full((16,), 11, jnp.int32)]
            kyl = wu[jnp.full((16,), 12, jnp.int32)]
            kyh = wu[jnp.full((16,), 13, jnp.int32)]
            kar = wu[jnp.full((16,), 14, jnp.int32)]

            koff = k - base

            @pl.when(k // _SHARD == wid)
            def _kill():
                cb = (koff // 16) * 16
                lane_in = koff - cb
                wchunk = s_v[pl.ds(cb, 16)]
                s_v[pl.ds(cb, 16)] = jnp.where(l16 == lane_in, -1.0, wchunk)

            mv_new = jnp.full((16,), -2.0, jnp.float32)
            iv_new = jnp.zeros((16,), jnp.int32)
            for c in range(_CHUNKS):
                sl = pl.ds(c * 16, 16)
                xx1 = jnp.maximum(x1_v[sl], kx1)
                yy1 = jnp.maximum(yl_v[sl], kyl)
                xx2 = jnp.minimum(x2_v[sl], kx2)
                yy2 = jnp.minimum(yh_v[sl], kyh)
                inter = (jnp.maximum(xx2 - xx1, 0.0)
                         * jnp.maximum(yy2 - yy1, 0.0))
                iou = inter / (ar_v[sl] + kar - inter)
                lane = l16 + (base + c * 16)
                s_new = jnp.where(iou > _IOU_T, -1.0, s_v[sl])
                iv_new = jnp.where(s_new > mv_new, lane, iv_new)
                mv_new = jnp.maximum(mv_new, s_new)
                s_v[sl] = s_new

            @pl.when(wid == 0)
            def _emit():
                out_v[t, :] = wu

            return (k0r, mv_new, iv_new)

        lax.fori_loop(0, _TOP_K, body, (zf, mv0, iv0))

        @pl.when(wid == 0)
        def _flush():
            pltpu.sync_copy(out_v, out_hbm)


def kernel(x, y, anchors):
    xf = x[0, :, 0]
    yy = y[0]

    def prep(a):
        return jnp.pad(a, (0, _PAD))

    feats = jnp.concatenate([
        prep(xf),
        prep(yy[:, 0]), prep(yy[:, 1]), prep(yy[:, 2]), prep(yy[:, 3]),
        prep(yy[:, 4]), prep(yy[:, 5]), prep(yy[:, 8]), prep(yy[:, 9]),
        prep(anchors[:, 0]), prep(anchors[:, 1]),
    ])

    mesh = plsc.VectorSubcoreMesh(core_axis_name="c", subcore_axis_name="s")
    run = functools.partial(
        pl.kernel,
        out_type=jax.ShapeDtypeStruct((_TOP_K, 16), jnp.float32),
        mesh=mesh,
        scratch_types=[
            pltpu.VMEM((11 * _SHARD,), jnp.float32),
            pltpu.VMEM((_WPAD,), jnp.float32),
            pltpu.VMEM((_WPAD,), jnp.float32),
            pltpu.VMEM((_WPAD,), jnp.float32),
            pltpu.VMEM((_WPAD,), jnp.float32),
            pltpu.VMEM((_WPAD,), jnp.float32),
            pltpu.VMEM((_WPAD,), jnp.float32),
            pltpu.VMEM((_WPAD,), jnp.float32),
            pltpu.VMEM((_WPAD,), jnp.float32),
            pltpu.VMEM((_WPAD,), jnp.float32),
            pltpu.VMEM((_WPAD,), jnp.float32),
            pltpu.VMEM((_WPAD,), jnp.float32),
            pltpu.VMEM((_WPAD,), jnp.float32),
            pltpu.VMEM((_WPAD,), jnp.float32),
            pltpu.VMEM((16,), jnp.float32),
            pltpu.VMEM((_GRID,), jnp.float32),
            pltpu.VMEM((_TOP_K, 16), jnp.float32),
            pltpu.VMEM_SHARED((2 * _GRID,), jnp.float32),
        ],
    )(_sc_nms)
    out = run(feats)
    return out[:, 2:10]


# speedup vs baseline: 1.0035x; 1.0035x over previous
import functools

import jax
import jax.numpy as jnp
from jax import lax
from jax.experimental import pallas as pl
from jax.experimental.pallas import tpu as pltpu
from jax.experimental.pallas import tpu_sc as plsc

_N = 5000
_TOP_K = 100
_SCALE = 192.0
_IOU_T = 0.3
_P = 5120
_PAD = _P - _N
_NS = 16
_SHARD = _P // _NS
_CHUNKS = _SHARD // 16
_WPAD = _SHARD + 16
_GRID = _NS * 16
_BIG = 2 ** 30


def _sc_nms(feat_hbm, out_hbm,
            f_v, s_v, cx_v, cy_v, w_v, p0x_v, p0y_v, p2x_v, p2y_v,
            x1_v, x2_v, yl_v, yh_v, ar_v,
            stage_v, grid_v, out_v, grid_sh):
    cid = lax.axis_index("c")
    wid = lax.axis_index("s")

    @pl.when(cid == 0)
    def _body():
        base = wid * _SHARD

        for j in range(11):
            pltpu.sync_copy(feat_hbm.at[pl.ds(j * _P + base, _SHARD)],
                            f_v.at[pl.ds(j * _SHARD, _SHARD)])

        mv0 = jnp.full((16,), -2.0, jnp.float32)
        iv0 = jnp.zeros((16,), jnp.int32)
        for c in range(_CHUNKS):
            sl = pl.ds(c * 16, 16)
            lane = lax.iota(jnp.int32, 16) + (base + c * 16)
            lg = f_v[pl.ds(0 * _SHARD + c * 16, 16)]
            sc = 1.0 / (1.0 + jnp.exp(-lg))
            ax = f_v[pl.ds(9 * _SHARD + c * 16, 16)]
            ay = f_v[pl.ds(10 * _SHARD + c * 16, 16)]
            cx = f_v[pl.ds(1 * _SHARD + c * 16, 16)] / _SCALE + ax
            cy = f_v[pl.ds(2 * _SHARD + c * 16, 16)] / _SCALE + ay
            w = f_v[pl.ds(3 * _SHARD + c * 16, 16)] / _SCALE
            h = f_v[pl.ds(4 * _SHARD + c * 16, 16)] / _SCALE
            half_w = w * 0.5
            half_h = h * 0.5
            x1 = cx - half_w
            x2 = cx + half_w
            yl = cy - half_h
            yh = cy + half_h
            s0 = jnp.where(lane < _N, sc, -1.0)
            iv0 = jnp.where(s0 > mv0, lane, iv0)
            mv0 = jnp.maximum(mv0, s0)
            s_v[sl] = s0
            cx_v[sl] = cx
            cy_v[sl] = cy
            w_v[sl] = w
            p0x_v[sl] = f_v[pl.ds(5 * _SHARD + c * 16, 16)] / _SCALE + ax
            p0y_v[sl] = f_v[pl.ds(6 * _SHARD + c * 16, 16)] / _SCALE + ay
            p2x_v[sl] = f_v[pl.ds(7 * _SHARD + c * 16, 16)] / _SCALE + ax
            p2y_v[sl] = f_v[pl.ds(8 * _SHARD + c * 16, 16)] / _SCALE + ay
            x1_v[sl] = x1
            x2_v[sl] = x2
            yl_v[sl] = yl
            yh_v[sl] = yh
            ar_v[sl] = (x2 - x1) * (yh - yl)

        l16 = lax.iota(jnp.int32, 16)
        zf = jnp.zeros((16,), jnp.float32)
        zi = jnp.zeros((16,), jnp.int32)

        def _vsel(cond_scalar_f32, a, b):
            return a * cond_scalar_f32 + b * (1.0 - cond_scalar_f32)

        def body(t, carry):
            k0r, mv, iv = carry
            mvv, ivv = mv, iv
            for sh in (1, 2, 4, 8):
                perm = l16 ^ sh
                pm = mvv[perm]
                pi = ivv[perm]
                b = (pm > mvv) | ((pm == mvv) & (pi < ivv))
                mvv = jnp.where(b, pm, mvv)
                ivv = jnp.where(b, pi, ivv)
            gi = ivv[0]
            off = jnp.clip(gi - base, 0, _SHARD - 1)

            gidx_f = (base + off).astype(jnp.float32)
            srcs = (cx_v, cy_v, w_v, p0x_v, p0y_v, p2x_v, p2y_v,
                    x1_v, x2_v, yl_v, yh_v, ar_v)
            stage = jnp.where(l16 == 1, gidx_f, mvv)
            stage = jnp.where(l16 >= 3, 0.0, stage)
            for j, ref in enumerate(srcs):
                stage = jnp.where(l16 == j + 3, ref[pl.ds(off, 16)][0], stage)
            stage_v[...] = stage
            par = (t % 2) * _GRID
            pltpu.sync_copy(stage_v, grid_sh.at[pl.ds(par + wid * 16, 16)])
            plsc.subcore_barrier()
            pltpu.sync_copy(grid_sh.at[pl.ds(par, _GRID)], grid_v)

            items = [(grid_v[pl.ds(j * 16, 16)][0], jnp.int32(j * 16))
                     for j in range(_NS)]
            while len(items) > 1:
                nxt = []
                for (ka, ra), (kb, rb) in zip(items[::2], items[1::2]):
                    b = kb > ka
                    nxt.append((jnp.maximum(ka, kb),
                                jnp.where(b, rb, ra)))
                items = nxt
            _wkey, rwin = items[0]
            w_rec = grid_v[pl.ds(rwin, 16)]

            has_f = (w_rec[0] >= 0.0).astype(jnp.float32)
            t0_f = (t == 0).astype(jnp.float32)
            k0r = _vsel(t0_f, w_rec, k0r)
            wu = _vsel(has_f, w_rec, k0r)

            k = wu[1].astype(jnp.int32)
            kx1 = wu[10]
            kx2 = wu[11]
            kyl = wu[12]
            kyh = wu[13]
            kar = wu[14]

            koff = k - base

            @pl.when(k // _SHARD == wid)
            def _kill():
                cb = (koff // 16) * 16
                lane_in = koff - cb
                wchunk = s_v[pl.ds(cb, 16)]
                s_v[pl.ds(cb, 16)] = jnp.where(l16 == lane_in, -1.0, wchunk)

            mv_new = jnp.full((16,), -2.0, jnp.float32)
            iv_new = jnp.zeros((16,), jnp.int32)
            for c in range(_CHUNKS):
                sl = pl.ds(c * 16, 16)
                xx1 = jnp.maximum(x1_v[sl], kx1)
                yy1 = jnp.maximum(yl_v[sl], kyl)
                xx2 = jnp.minimum(x2_v[sl], kx2)
                yy2 = jnp.minimum(yh_v[sl], kyh)
                inter = (jnp.maximum(xx2 - xx1, 0.0)
                         * jnp.maximum(yy2 - yy1, 0.0))
                iou = inter / (ar_v[sl] + kar - inter)
                lane = l16 + (base + c * 16)
                s_new = jnp.where(iou > _IOU_T, -1.0, s_v[sl])
                iv_new = jnp.where(s_new > mv_new, lane, iv_new)
                mv_new = jnp.maximum(mv_new, s_new)
                s_v[sl] = s_new

            @pl.when(wid == 0)
            def _emit():
                out_v[t, :] = wu

            return (k0r, mv_new, iv_new)

        lax.fori_loop(0, _TOP_K, body, (zf, mv0, iv0))

        @pl.when(wid == 0)
        def _flush():
            pltpu.sync_copy(out_v, out_hbm)


def kernel(x, y, anchors):
    xf = x[0, :, 0]
    yy = y[0]

    def prep(a):
        return jnp.pad(a, (0, _PAD))

    feats = jnp.concatenate([
        prep(xf),
        prep(yy[:, 0]), prep(yy[:, 1]), prep(yy[:, 2]), prep(yy[:, 3]),
        prep(yy[:, 4]), prep(yy[:, 5]), prep(yy[:, 8]), prep(yy[:, 9]),
        prep(anchors[:, 0]), prep(anchors[:, 1]),
    ])

    mesh = plsc.VectorSubcoreMesh(core_axis_name="c", subcore_axis_name="s")
    run = functools.partial(
        pl.kernel,
        out_type=jax.ShapeDtypeStruct((_TOP_K, 16), jnp.float32),
        mesh=mesh,
        scratch_types=[
            pltpu.VMEM((11 * _SHARD,), jnp.float32),
            pltpu.VMEM((_WPAD,), jnp.float32),
            pltpu.VMEM((_WPAD,), jnp.float32),
            pltpu.VMEM((_WPAD,), jnp.float32),
            pltpu.VMEM((_WPAD,), jnp.float32),
            pltpu.VMEM((_WPAD,), jnp.float32),
            pltpu.VMEM((_WPAD,), jnp.float32),
            pltpu.VMEM((_WPAD,), jnp.float32),
            pltpu.VMEM((_WPAD,), jnp.float32),
            pltpu.VMEM((_WPAD,), jnp.float32),
            pltpu.VMEM((_WPAD,), jnp.float32),
            pltpu.VMEM((_WPAD,), jnp.float32),
            pltpu.VMEM((_WPAD,), jnp.float32),
            pltpu.VMEM((_WPAD,), jnp.float32),
            pltpu.VMEM((16,), jnp.float32),
            pltpu.VMEM((_GRID,), jnp.float32),
            pltpu.VMEM((_TOP_K, 16), jnp.float32),
            pltpu.VMEM_SHARED((2 * _GRID,), jnp.float32),
        ],
    )(_sc_nms)
    out = run(feats)
    return out[:, 2:10]


# speedup vs baseline: 1.0549x; 1.0512x over previous
import functools

import jax
import jax.numpy as jnp
from jax import lax
from jax.experimental import pallas as pl
from jax.experimental.pallas import tpu as pltpu
from jax.experimental.pallas import tpu_sc as plsc

_N = 5000
_TOP_K = 100
_SCALE = 192.0
_IOU_T = 0.3
_P = 5120
_PAD = _P - _N
_NS = 16
_SHARD = _P // _NS
_CHUNKS = _SHARD // 16
_WPAD = _SHARD + 16
_GRID = _NS * 16
_BIG = 2 ** 30


def _sc_nms(feat_hbm, out_hbm,
            f_v, s_v, cx_v, cy_v, w_v, p0x_v, p0y_v, p2x_v, p2y_v,
            x1_v, x2_v, yl_v, yh_v, ar_v,
            stage_v, grid_v, out_v, grid_sh):
    cid = lax.axis_index("c")
    wid = lax.axis_index("s")

    @pl.when(cid == 0)
    def _body():
        base = wid * _SHARD

        for j in range(11):
            pltpu.sync_copy(feat_hbm.at[pl.ds(j * _P + base, _SHARD)],
                            f_v.at[pl.ds(j * _SHARD, _SHARD)])

        mv0 = jnp.full((16,), -2.0, jnp.float32)
        iv0 = jnp.zeros((16,), jnp.int32)
        for c in range(_CHUNKS):
            sl = pl.ds(c * 16, 16)
            lane = lax.iota(jnp.int32, 16) + (base + c * 16)
            lg = f_v[pl.ds(0 * _SHARD + c * 16, 16)]
            sc = 1.0 / (1.0 + jnp.exp(-lg))
            ax = f_v[pl.ds(9 * _SHARD + c * 16, 16)]
            ay = f_v[pl.ds(10 * _SHARD + c * 16, 16)]
            cx = f_v[pl.ds(1 * _SHARD + c * 16, 16)] / _SCALE + ax
            cy = f_v[pl.ds(2 * _SHARD + c * 16, 16)] / _SCALE + ay
            w = f_v[pl.ds(3 * _SHARD + c * 16, 16)] / _SCALE
            h = f_v[pl.ds(4 * _SHARD + c * 16, 16)] / _SCALE
            half_w = w * 0.5
            half_h = h * 0.5
            x1 = cx - half_w
            x2 = cx + half_w
            yl = cy - half_h
            yh = cy + half_h
            s0 = jnp.where(lane < _N, sc, -1.0)
            iv0 = jnp.where(s0 > mv0, lane, iv0)
            mv0 = jnp.maximum(mv0, s0)
            s_v[sl] = s0
            cx_v[sl] = cx
            cy_v[sl] = cy
            w_v[sl] = w
            p0x_v[sl] = f_v[pl.ds(5 * _SHARD + c * 16, 16)] / _SCALE + ax
            p0y_v[sl] = f_v[pl.ds(6 * _SHARD + c * 16, 16)] / _SCALE + ay
            p2x_v[sl] = f_v[pl.ds(7 * _SHARD + c * 16, 16)] / _SCALE + ax
            p2y_v[sl] = f_v[pl.ds(8 * _SHARD + c * 16, 16)] / _SCALE + ay
            x1_v[sl] = x1
            x2_v[sl] = x2
            yl_v[sl] = yl
            yh_v[sl] = yh
            ar_v[sl] = (x2 - x1) * (yh - yl)

        l16 = lax.iota(jnp.int32, 16)
        zf = jnp.zeros((16,), jnp.float32)
        zi = jnp.zeros((16,), jnp.int32)

        def _vsel(cond_scalar_f32, a, b):
            return a * cond_scalar_f32 + b * (1.0 - cond_scalar_f32)

        def body(t, carry):
            k0r, mv, iv = carry
            mvv, ivv = mv, iv
            for sh in (1, 2, 4, 8):
                perm = l16 ^ sh
                pm = mvv[perm]
                pi = ivv[perm]
                b = (pm > mvv) | ((pm == mvv) & (pi < ivv))
                mvv = jnp.where(b, pm, mvv)
                ivv = jnp.where(b, pi, ivv)
            m = mvv[0]
            gi = ivv[0]
            off = jnp.clip(gi - base, 0, _SHARD - 1)

            gidx_f = (base + off).astype(jnp.float32)
            srcs = (cx_v, cy_v, w_v, p0x_v, p0y_v, p2x_v, p2y_v,
                    x1_v, x2_v, yl_v, yh_v, ar_v)
            stage = jnp.where(l16 == 0, m, jnp.where(l16 == 1, gidx_f, 0.0))
            stage = jnp.where(l16 == 2, m, stage)
            for j, ref in enumerate(srcs):
                stage = jnp.where(l16 == j + 3, ref[pl.ds(off, 16)][0], stage)
            stage_v[...] = stage
            par = (t % 2) * _GRID
            pltpu.sync_copy(stage_v, grid_sh.at[pl.ds(par + wid * 16, 16)])
            plsc.subcore_barrier()
            pltpu.sync_copy(grid_sh.at[pl.ds(par, _GRID)], grid_v)

            items = [(grid_v[pl.ds(j * 16, 16)][0], jnp.int32(j * 16))
                     for j in range(_NS)]
            while len(items) > 1:
                nxt = []
                for (ka, ra), (kb, rb) in zip(items[::2], items[1::2]):
                    b = kb > ka
                    nxt.append((jnp.maximum(ka, kb),
                                jnp.where(b, rb, ra)))
                items = nxt
            _wkey, rwin = items[0]
            w_rec = grid_v[pl.ds(rwin, 16)]

            has_f = (w_rec[0] >= 0.0).astype(jnp.float32)
            t0_f = (t == 0).astype(jnp.float32)
            k0r = _vsel(t0_f, w_rec, k0r)
            wu = _vsel(has_f, w_rec, k0r)

            k = wu[1].astype(jnp.int32)
            kx1 = wu[10]
            kx2 = wu[11]
            kyl = wu[12]
            kyh = wu[13]
            kar = wu[14]

            mv_new = jnp.full((16,), -2.0, jnp.float32)
            iv_new = jnp.zeros((16,), jnp.int32)
            for c in range(_CHUNKS):
                sl = pl.ds(c * 16, 16)
                xx1 = jnp.maximum(x1_v[sl], kx1)
                yy1 = jnp.maximum(yl_v[sl], kyl)
                xx2 = jnp.minimum(x2_v[sl], kx2)
                yy2 = jnp.minimum(yh_v[sl], kyh)
                inter = (jnp.maximum(xx2 - xx1, 0.0)
                         * jnp.maximum(yy2 - yy1, 0.0))
                iou = inter / (ar_v[sl] + kar - inter)
                lane = l16 + (base + c * 16)
                cond = (iou > _IOU_T) | (lane == k)
                s_new = jnp.where(cond, -1.0, s_v[sl])
                iv_new = jnp.where(s_new > mv_new, lane, iv_new)
                mv_new = jnp.maximum(mv_new, s_new)
                s_v[sl] = s_new

            @pl.when(wid == 0)
            def _emit():
                out_v[t, :] = wu

            return (k0r, mv_new, iv_new)

        lax.fori_loop(0, _TOP_K, body, (zf, mv0, iv0))

        @pl.when(wid == 0)
        def _flush():
            pltpu.sync_copy(out_v, out_hbm)


def kernel(x, y, anchors):
    xf = x[0, :, 0]
    yy = y[0]

    def prep(a):
        return jnp.pad(a, (0, _PAD))

    feats = jnp.concatenate([
        prep(xf),
        prep(yy[:, 0]), prep(yy[:, 1]), prep(yy[:, 2]), prep(yy[:, 3]),
        prep(yy[:, 4]), prep(yy[:, 5]), prep(yy[:, 8]), prep(yy[:, 9]),
        prep(anchors[:, 0]), prep(anchors[:, 1]),
    ])

    mesh = plsc.VectorSubcoreMesh(core_axis_name="c", subcore_axis_name="s")
    run = functools.partial(
        pl.kernel,
        out_type=jax.ShapeDtypeStruct((_TOP_K, 16), jnp.float32),
        mesh=mesh,
        scratch_types=[
            pltpu.VMEM((11 * _SHARD,), jnp.float32),
            pltpu.VMEM((_WPAD,), jnp.float32),
            pltpu.VMEM((_WPAD,), jnp.float32),
            pltpu.VMEM((_WPAD,), jnp.float32),
            pltpu.VMEM((_WPAD,), jnp.float32),
            pltpu.VMEM((_WPAD,), jnp.float32),
            pltpu.VMEM((_WPAD,), jnp.float32),
            pltpu.VMEM((_WPAD,), jnp.float32),
            pltpu.VMEM((_WPAD,), jnp.float32),
            pltpu.VMEM((_WPAD,), jnp.float32),
            pltpu.VMEM((_WPAD,), jnp.float32),
            pltpu.VMEM((_WPAD,), jnp.float32),
            pltpu.VMEM((_WPAD,), jnp.float32),
            pltpu.VMEM((_WPAD,), jnp.float32),
            pltpu.VMEM((16,), jnp.float32),
            pltpu.VMEM((_GRID,), jnp.float32),
            pltpu.VMEM((_TOP_K, 16), jnp.float32),
            pltpu.VMEM_SHARED((2 * _GRID,), jnp.float32),
        ],
    )(_sc_nms)
    out = run(feats)
    return out[:, 2:10]


# speedup vs baseline: 1.0567x; 1.0018x over previous
import functools

import jax
import jax.numpy as jnp
from jax import lax
from jax.experimental import pallas as pl
from jax.experimental.pallas import tpu as pltpu
from jax.experimental.pallas import tpu_sc as plsc

_N = 5000
_TOP_K = 100
_SCALE = 192.0
_IOU_T = 0.3
_P = 5120
_PAD = _P - _N
_NS = 16
_SHARD = _P // _NS
_CHUNKS = _SHARD // 16
_WPAD = _SHARD + 16
_GRID = _NS * 16


def _sc_nms(feat_hbm, out_hbm,
            f_v, s_v, cx_v, cy_v, w_v, p0x_v, p0y_v, p2x_v, p2y_v,
            x1_v, x2_v, yl_v, yh_v, ar_v,
            stage_v, grid_v, out_v, grid_sh):
    cid = lax.axis_index("c")
    wid = lax.axis_index("s")

    @pl.when(cid == 0)
    def _body():
        base = wid * _SHARD

        for j in range(11):
            pltpu.sync_copy(feat_hbm.at[pl.ds(j * _P + base, _SHARD)],
                            f_v.at[pl.ds(j * _SHARD, _SHARD)])

        mv0 = jnp.full((16,), -2.0, jnp.float32)
        iv0 = jnp.zeros((16,), jnp.int32)
        for c in range(_CHUNKS):
            sl = pl.ds(c * 16, 16)
            lane = lax.iota(jnp.int32, 16) + (base + c * 16)
            lg = f_v[pl.ds(0 * _SHARD + c * 16, 16)]
            sc = 1.0 / (1.0 + jnp.exp(-lg))
            ax = f_v[pl.ds(9 * _SHARD + c * 16, 16)]
            ay = f_v[pl.ds(10 * _SHARD + c * 16, 16)]
            cx = f_v[pl.ds(1 * _SHARD + c * 16, 16)] / _SCALE + ax
            cy = f_v[pl.ds(2 * _SHARD + c * 16, 16)] / _SCALE + ay
            w = f_v[pl.ds(3 * _SHARD + c * 16, 16)] / _SCALE
            h = f_v[pl.ds(4 * _SHARD + c * 16, 16)] / _SCALE
            half_w = w * 0.5
            half_h = h * 0.5
            x1 = cx - half_w
            x2 = cx + half_w
            yl = cy - half_h
            yh = cy + half_h
            s0 = jnp.where(lane < _N, sc, -1.0)
            iv0 = jnp.where(s0 > mv0, lane, iv0)
            mv0 = jnp.maximum(mv0, s0)
            s_v[sl] = s0
            cx_v[sl] = cx
            cy_v[sl] = cy
            w_v[sl] = w
            p0x_v[sl] = f_v[pl.ds(5 * _SHARD + c * 16, 16)] / _SCALE + ax
            p0y_v[sl] = f_v[pl.ds(6 * _SHARD + c * 16, 16)] / _SCALE + ay
            p2x_v[sl] = f_v[pl.ds(7 * _SHARD + c * 16, 16)] / _SCALE + ax
            p2y_v[sl] = f_v[pl.ds(8 * _SHARD + c * 16, 16)] / _SCALE + ay
            x1_v[sl] = x1
            x2_v[sl] = x2
            yl_v[sl] = yl
            yh_v[sl] = yh
            ar_v[sl] = (x2 - x1) * (yh - yl)

        l16 = lax.iota(jnp.int32, 16)
        zf = jnp.zeros((16,), jnp.float32)

        def _vsel(cond_scalar_f32, a, b):
            return a * cond_scalar_f32 + b * (1.0 - cond_scalar_f32)

        def body(t, carry):
            k0r, mv, iv = carry
            mvv, ivv = mv, iv
            for sh in (1, 2, 4, 8):
                perm = l16 ^ sh
                pm = mvv[perm]
                pi = ivv[perm]
                b = (pm > mvv) | ((pm == mvv) & (pi < ivv))
                mvv = jnp.where(b, pm, mvv)
                ivv = jnp.where(b, pi, ivv)
            m = mvv[0]
            gi = ivv[0]
            off = jnp.clip(gi - base, 0, _SHARD - 1)

            gidx_f = (base + off).astype(jnp.float32)
            srcs = (cx_v, cy_v, w_v, p0x_v, p0y_v, p2x_v, p2y_v,
                    x1_v, x2_v, yl_v, yh_v, ar_v)
            stage = jnp.where(l16 == 0, m, jnp.where(l16 == 1, gidx_f, 0.0))
            stage = jnp.where(l16 == 2, m, stage)
            for j, ref in enumerate(srcs):
                stage = jnp.where(l16 == j + 3, ref[pl.ds(off, 16)][0], stage)
            stage_v[...] = stage
            par = (t % 2) * _GRID
            pltpu.sync_copy(stage_v, grid_sh.at[pl.ds(par + wid * 16, 16)])
            plsc.subcore_barrier()
            pltpu.sync_copy(grid_sh.at[pl.ds(par, _GRID)], grid_v)

            items = [(grid_v[pl.ds(j * 16, 16)][0], jnp.int32(j * 16))
                     for j in range(_NS)]
            while len(items) > 1:
                nxt = []
                for (ka, ra), (kb, rb) in zip(items[::2], items[1::2]):
                    b = kb > ka
                    nxt.append((jnp.maximum(ka, kb),
                                jnp.where(b, rb, ra)))
                items = nxt
            _wkey, rwin = items[0]
            w_rec = grid_v[pl.ds(rwin, 16)]

            has_f = (w_rec[0] >= 0.0).astype(jnp.float32)
            t0_f = (t == 0).astype(jnp.float32)
            k0r = _vsel(t0_f, w_rec, k0r)
            wu = _vsel(has_f, w_rec, k0r)

            k = wu[1].astype(jnp.int32)
            kx1 = wu[10]
            kx2 = wu[11]
            kyl = wu[12]
            kyh = wu[13]
            kar = wu[14]

            mv_new = jnp.full((16,), -2.0, jnp.float32)
            iv_new = jnp.zeros((16,), jnp.int32)
            for c in range(_CHUNKS):
                sl = pl.ds(c * 16, 16)
                xx1 = jnp.maximum(x1_v[sl], kx1)
                yy1 = jnp.maximum(yl_v[sl], kyl)
                xx2 = jnp.minimum(x2_v[sl], kx2)
                yy2 = jnp.minimum(yh_v[sl], kyh)
                inter = (jnp.maximum(xx2 - xx1, 0.0)
                         * jnp.maximum(yy2 - yy1, 0.0))
                iou = inter / (ar_v[sl] + kar - inter)
                lane = l16 + (base + c * 16)
                cond = (iou > _IOU_T) | (lane == k)
                s_new = jnp.where(cond, -1.0, s_v[sl])
                iv_new = jnp.where(s_new > mv_new, lane, iv_new)
                mv_new = jnp.maximum(mv_new, s_new)
                s_v[sl] = s_new

            @pl.when(wid == 0)
            def _emit():
                out_v[t, :] = wu

            return (k0r, mv_new, iv_new)

        lax.fori_loop(0, _TOP_K, body, (zf, mv0, iv0))

        @pl.when(wid == 0)
        def _flush():
            pltpu.sync_copy(out_v, out_hbm)


def kernel(x, y, anchors):
    xf = x[0, :, 0]
    yy = y[0]

    def prep(a):
        return jnp.pad(a, (0, _PAD))

    feats = jnp.concatenate([
        prep(xf),
        prep(yy[:, 0]), prep(yy[:, 1]), prep(yy[:, 2]), prep(yy[:, 3]),
        prep(yy[:, 4]), prep(yy[:, 5]), prep(yy[:, 8]), prep(yy[:, 9]),
        prep(anchors[:, 0]), prep(anchors[:, 1]),
    ])

    mesh = plsc.VectorSubcoreMesh(core_axis_name="c", subcore_axis_name="s")
    run = functools.partial(
        pl.kernel,
        out_type=jax.ShapeDtypeStruct((_TOP_K, 16), jnp.float32),
        mesh=mesh,
        scratch_types=[
            pltpu.VMEM((11 * _SHARD,), jnp.float32),
            pltpu.VMEM((_WPAD,), jnp.float32),
            pltpu.VMEM((_WPAD,), jnp.float32),
            pltpu.VMEM((_WPAD,), jnp.float32),
            pltpu.VMEM((_WPAD,), jnp.float32),
            pltpu.VMEM((_WPAD,), jnp.float32),
            pltpu.VMEM((_WPAD,), jnp.float32),
            pltpu.VMEM((_WPAD,), jnp.float32),
            pltpu.VMEM((_WPAD,), jnp.float32),
            pltpu.VMEM((_WPAD,), jnp.float32),
            pltpu.VMEM((_WPAD,), jnp.float32),
            pltpu.VMEM((_WPAD,), jnp.float32),
            pltpu.VMEM((_WPAD,), jnp.float32),
            pltpu.VMEM((_WPAD,), jnp.float32),
            pltpu.VMEM((16,), jnp.float32),
            pltpu.VMEM((_GRID,), jnp.float32),
            pltpu.VMEM((_TOP_K, 16), jnp.float32),
            pltpu.VMEM_SHARED((2 * _GRID,), jnp.float32),
        ],
    )(_sc_nms)
    out = run(feats)
    return out[:, 2:10]
